# Initial kernel scaffold; baseline (speedup 1.0000x reference)
import jax
import jax.numpy as jnp
from jax.experimental import pallas as pl


def kernel(input_ids, token_type_ids, word_emb, type_emb, gamma, beta):
    raise NotImplementedError("write your pallas kernel here")


# speedup vs baseline: 2.3380x; 2.3380x over previous
import functools

import jax
import jax.numpy as jnp
from jax import lax
from jax.experimental import pallas as pl
from jax.experimental.pallas import tpu as pltpu
from jax.experimental.pallas import tpu_sc as plsc

_EPS = 1e-12
_L = 16
_NC = 2
_NS = 16
_NW = _NC * _NS
_GCHUNK = 128


def _bcast(x, dtype=jnp.float32):
  return lax.broadcast(x, (_L,)).astype(dtype)


def _rsqrt_vec(v):
  vi = lax.bitcast_convert_type(v, jnp.int32)
  seed_i = jnp.int32(0x5F3759DF) - lax.shift_right_arithmetic(vi, 1)
  y = lax.bitcast_convert_type(seed_i, jnp.float32)
  half = v * 0.5
  for _ in range(3):
    y = y * (1.5 - half * y * y)
  return y


def _make_kernel(N, D, V):
  n_per_w = N // _NW
  n_groups = n_per_w // _GCHUNK
  kd = D // _L
  inv_d = 1.0 / D

  mesh = plsc.VectorSubcoreMesh(
      core_axis_name="c", subcore_axis_name="s",
      num_cores=_NC, num_subcores=_NS)

  @functools.partial(
      pl.kernel,
      out_type=jax.ShapeDtypeStruct((N, D), jnp.float32),
      mesh=mesh,
      compiler_params=pltpu.CompilerParams(needs_layout_passes=False),
      scratch_types=[
          pltpu.VMEM((n_groups, _GCHUNK), jnp.int32),
          pltpu.VMEM((n_per_w,), jnp.int32),
          pltpu.VMEM((n_per_w, D), jnp.float32),
          pltpu.VMEM((2, D), jnp.float32),
          pltpu.VMEM((D,), jnp.float32),
          pltpu.VMEM((D,), jnp.float32),
          pltpu.SemaphoreType.DMA,
      ],
  )
  def emb_ln(ids_hbm, tt_hbm, word_hbm, te_hbm, g_hbm, b_hbm, out_hbm,
             idx_v, tt_v, rows_v, te_v, g_v, b_v, sem):
    wid = lax.axis_index("s") * _NC + lax.axis_index("c")
    base = wid * n_per_w

    pltpu.sync_copy(ids_hbm.at[wid], idx_v)
    pltpu.sync_copy(tt_hbm.at[wid], tt_v)
    pltpu.sync_copy(te_hbm, te_v)
    pltpu.sync_copy(g_hbm, g_v)
    pltpu.sync_copy(b_hbm, b_v)

    copies = [
        pltpu.async_copy(
            word_hbm.at[idx_v.at[c]],
            rows_v.at[pl.ds(c * _GCHUNK, _GCHUNK)],
            sem,
        )
        for c in range(n_groups)
    ]
    for cp in copies:
      cp.wait()

    r0 = [te_v[0, pl.ds(k * _L, _L)] for k in range(kd)]
    r1 = [te_v[1, pl.ds(k * _L, _L)] for k in range(kd)]
    dl = [a - b for a, b in zip(r1, r0)]
    gv = [g_v[pl.ds(k * _L, _L)] for k in range(kd)]
    bv = [b_v[pl.ds(k * _L, _L)] for k in range(kd)]

    def body(g, carry):
      tv = tt_v[pl.ds(g * _L, _L)].astype(jnp.float32)
      for j in range(_L):
        i = g * _L + j
        t = lax.broadcast(tv[j], (_L,))
        x = [rows_v[i, pl.ds(k * _L, _L)] for k in range(kd)]
        y = [x[k] + (r0[k] + t * dl[k]) for k in range(kd)]
        s = y[0] + y[1]
        for k in range(2, kd):
          s = s + y[k]
        q = y[0] * y[0] + y[1] * y[1]
        for k in range(2, kd):
          q = q + y[k] * y[k]
        sv = _bcast(jnp.sum(s))
        qv = _bcast(jnp.sum(q))
        mu = sv * inv_d
        var = qv * inv_d - mu * mu
        rstd = _rsqrt_vec(var + _EPS)
        for k in range(kd):
          rows_v[i, pl.ds(k * _L, _L)] = (y[k] - mu) * rstd * gv[k] + bv[k]
      return carry

    lax.fori_loop(0, n_per_w // _L, body, 0)

    pltpu.sync_copy(rows_v, out_hbm.at[pl.ds(base, n_per_w)])

  return emb_ln


@jax.jit
def kernel(input_ids, token_type_ids, word_emb, type_emb, gamma, beta):
  B, S = input_ids.shape
  V, D = word_emb.shape
  N = B * S
  n_per_w = N // _NW
  ids = input_ids.reshape(_NW, n_per_w // _GCHUNK, _GCHUNK).astype(jnp.int32)
  tt = token_type_ids.reshape(_NW, n_per_w).astype(jnp.int32)
  out = _make_kernel(N, D, V)(
      ids, tt, word_emb, type_emb, gamma, beta)
  return out.reshape(B, S, D)
